# baseline (speedup 1.0000x reference)
import functools

import jax
import jax.numpy as jnp
from jax import lax
from jax.experimental import pallas as pl
from jax.experimental.pallas import tpu as pltpu
from jax.experimental.pallas import tpu_sc as plsc

N_POS = 1024
SUB = 128
SIZE = 768
NSEG = 6
LANES = 16


def _sc_workers():
    try:
        info = plsc.get_sparse_core_info()
        return info.num_cores, info.num_subcores
    except Exception:
        return 2, 16


def kernel(xmin, ymin, xmax, ymax, width, height, first_page, last_page,
           x_table, y_table, w_table, h_table,
           first_page_embedding, last_page_embedding):
    B, L = xmin.shape
    NB = B * L
    NC, NS = _sc_workers()
    NW = NC * NS
    rows_per_w = NB // NW
    R = 32
    NBUF = 4
    n_chunks = rows_per_w // R
    n_rounds = n_chunks // NBUF
    KPC = R * NSEG
    KW = rows_per_w * NSEG
    assert rows_per_w * NW == NB and n_rounds * NBUF * R == rows_per_w

    table = jnp.concatenate([x_table, y_table, w_table, h_table], axis=0)

    coords_i = jnp.stack(
        [a.reshape(NB) for a in (xmin, ymin, xmax, ymax, width, height)],
        axis=-1).reshape(NB * NSEG)
    coords_bits = lax.bitcast_convert_type(coords_i, jnp.int32)
    fp = first_page.reshape(NB)
    lp = last_page.reshape(NB)

    mesh = plsc.VectorSubcoreMesh(core_axis_name="c", subcore_axis_name="s",
                                  num_cores=NC, num_subcores=NS)

    @functools.partial(
        pl.kernel,
        out_type=jax.ShapeDtypeStruct((NB * NSEG, SUB), jnp.float32),
        mesh=mesh,
        scratch_types=[
            pltpu.VMEM((KW,), jnp.int32),
            pltpu.VMEM((rows_per_w + LANES,), jnp.float32),
            pltpu.VMEM((rows_per_w + LANES,), jnp.float32),
            pltpu.VMEM((NBUF, KPC, SUB), jnp.float32),
            pltpu.VMEM((2, SIZE), jnp.float32),
            pltpu.SemaphoreType.DMA,
            pltpu.SemaphoreType.DMA,
            pltpu.SemaphoreType.DMA,
            pltpu.SemaphoreType.DMA,
            pltpu.SemaphoreType.DMA,
            pltpu.SemaphoreType.DMA,
            pltpu.SemaphoreType.DMA,
            pltpu.SemaphoreType.DMA,
        ],
    )
    def sc_kernel(table_h, coords_h, fp_h, lp_h, fpe_h, lpe_h, out_h,
                  idx_v, fp_v, lp_v, rows_v, pe_v, *sems):
        gsems, osems = sems[:NBUF], sems[NBUF:]
        lane_iota = lax.broadcasted_iota(jnp.int32, (LANES,), 0)
        lane_bcast = [lane_iota * 0 + u for u in range(LANES)]
        wid = lax.axis_index("s") * NC + lax.axis_index("c")
        base_w = wid * rows_per_w
        pltpu.sync_copy(fpe_h, pe_v.at[0])
        pltpu.sync_copy(lpe_h, pe_v.at[1])
        pltpu.sync_copy(fp_h.at[pl.ds(base_w, rows_per_w)],
                        fp_v.at[pl.ds(0, rows_per_w)])
        pltpu.sync_copy(lp_h.at[pl.ds(base_w, rows_per_w)],
                        lp_v.at[pl.ds(0, rows_per_w)])
        pltpu.sync_copy(coords_h.at[pl.ds(base_w * NSEG, KW)], idx_v)

        def idx_body(t, _):
            sl = pl.ds(t * LANES, LANES)
            k = t * LANES + lane_iota
            s = k - 6 * ((k * 43691) >> 18)
            v = lax.bitcast_convert_type(idx_v[sl], jnp.float32)
            v = jnp.where(s == 5, v * 5.0, v)
            v = jnp.minimum(v * float(N_POS), float(N_POS - 1))
            off = jnp.where(s >= 4, s - 2, s & 1) * N_POS
            idx_v[sl] = v.astype(jnp.int32) + off
            return 0

        lax.fori_loop(0, KW // LANES, idx_body, 0)

        def compute_chunk(c, b):
            def row_body(i, _):
                fpi = fp_v[pl.ds(c * R + i, LANES)][0]
                lpi = lp_v[pl.ds(c * R + i, LANES)][0]
                for a in range(NSEG):
                    r = i * NSEG + a
                    for j in range(SUB // LANES):
                        sl = pl.ds(j * LANES, LANES)
                        pe = pl.ds(a * SUB + j * LANES, LANES)
                        rows_v[b, r, sl] = (rows_v[b, r, sl]
                                            + fpi * pe_v[0, pe]
                                            + lpi * pe_v[1, pe])
                return 0

            lax.fori_loop(0, R, row_body, 0)

        def round_body(m, carry):
            c0 = m * NBUF
            ghs = []
            for b in range(NBUF):
                k0 = (c0 + b) * KPC
                ghs.append(pltpu.async_copy(
                    table_h.at[idx_v.at[pl.ds(k0, KPC)]],
                    rows_v.at[b], gsems[b]))
            ohs = []
            for b in range(NBUF):
                ghs[b].wait()
                compute_chunk(c0 + b, b)
                k0 = (c0 + b) * KPC
                ohs.append(pltpu.async_copy(
                    rows_v.at[b],
                    out_h.at[pl.ds(base_w * NSEG + k0, KPC)], osems[b]))
            for oh in ohs:
                oh.wait()
            return carry

        lax.fori_loop(0, n_rounds, round_body, 0)

    out = sc_kernel(table, coords_bits, fp, lp,
                    first_page_embedding, last_page_embedding)
    return out.reshape(B, L, SIZE)


# speedup vs baseline: 2.4591x; 2.4591x over previous
import functools

import jax
import jax.numpy as jnp
from jax import lax
from jax.experimental import pallas as pl
from jax.experimental.pallas import tpu as pltpu
from jax.experimental.pallas import tpu_sc as plsc

N_POS = 1024
SUB = 128
SIZE = 768
NSEG = 6
LANES = 16


def _sc_workers():
    try:
        info = plsc.get_sparse_core_info()
        return info.num_cores, info.num_subcores
    except Exception:
        return 2, 16


def kernel(xmin, ymin, xmax, ymax, width, height, first_page, last_page,
           x_table, y_table, w_table, h_table,
           first_page_embedding, last_page_embedding):
    B, L = xmin.shape
    NB = B * L
    NC, NS = _sc_workers()
    NW = NC * NS
    rows_per_w = NB // NW
    R = 16
    NBUF = 4
    n_chunks = rows_per_w // R
    n_rounds = n_chunks // NBUF
    KPC = R * NSEG
    KW = rows_per_w * NSEG
    assert rows_per_w * NW == NB and n_rounds * NBUF * R == rows_per_w

    table = jnp.concatenate([x_table, y_table, w_table, h_table], axis=0)

    coords_i = jnp.stack(
        [a.reshape(NB) for a in (xmin, ymin, xmax, ymax, width, height)],
        axis=-1).reshape(NB * NSEG)
    coords_bits = lax.bitcast_convert_type(coords_i, jnp.int32)
    fp = first_page.reshape(NB)
    lp = last_page.reshape(NB)

    mesh = plsc.VectorSubcoreMesh(core_axis_name="c", subcore_axis_name="s",
                                  num_cores=NC, num_subcores=NS)

    @functools.partial(
        pl.kernel,
        out_type=jax.ShapeDtypeStruct((NB * NSEG, SUB), jnp.float32),
        mesh=mesh,
        scratch_types=[
            pltpu.VMEM((KW,), jnp.int32),
            pltpu.VMEM((rows_per_w + LANES,), jnp.float32),
            pltpu.VMEM((rows_per_w + LANES,), jnp.float32),
            pltpu.VMEM((NBUF, KPC, SUB), jnp.float32),
            pltpu.VMEM_SHARED((4 * N_POS, SUB), jnp.float32),
            pltpu.VMEM((2, SIZE), jnp.float32),
            pltpu.SemaphoreType.DMA,
            pltpu.SemaphoreType.DMA,
            pltpu.SemaphoreType.DMA,
            pltpu.SemaphoreType.DMA,
            pltpu.SemaphoreType.DMA,
            pltpu.SemaphoreType.DMA,
            pltpu.SemaphoreType.DMA,
            pltpu.SemaphoreType.DMA,
        ],
    )
    def sc_kernel(table_h, coords_h, fp_h, lp_h, fpe_h, lpe_h, out_h,
                  idx_v, fp_v, lp_v, rows_v, tab_sh, pe_v, *sems):
        gsems, osems = sems[:NBUF], sems[NBUF:]
        lane_iota = lax.broadcasted_iota(jnp.int32, (LANES,), 0)
        lane_bcast = [lane_iota * 0 + u for u in range(LANES)]
        sid = lax.axis_index("s")
        wid = sid * NC + lax.axis_index("c")
        base_w = wid * rows_per_w
        @pl.when(sid == 0)
        def _stage_table():
            pltpu.sync_copy(table_h, tab_sh)
        plsc.subcore_barrier()
        pltpu.sync_copy(fpe_h, pe_v.at[0])
        pltpu.sync_copy(lpe_h, pe_v.at[1])
        pltpu.sync_copy(fp_h.at[pl.ds(base_w, rows_per_w)],
                        fp_v.at[pl.ds(0, rows_per_w)])
        pltpu.sync_copy(lp_h.at[pl.ds(base_w, rows_per_w)],
                        lp_v.at[pl.ds(0, rows_per_w)])
        pltpu.sync_copy(coords_h.at[pl.ds(base_w * NSEG, KW)], idx_v)

        def idx_body(t, _):
            sl = pl.ds(t * LANES, LANES)
            k = t * LANES + lane_iota
            s = k - 6 * ((k * 43691) >> 18)
            v = lax.bitcast_convert_type(idx_v[sl], jnp.float32)
            v = jnp.where(s == 5, v * 5.0, v)
            v = jnp.minimum(v * float(N_POS), float(N_POS - 1))
            off = jnp.where(s >= 4, s - 2, s & 1) * N_POS
            idx_v[sl] = v.astype(jnp.int32) + off
            return 0

        lax.fori_loop(0, KW // LANES, idx_body, 0)

        def compute_chunk(c, b):
            def row_body(i, _):
                fpi = fp_v[pl.ds(c * R + i, LANES)][0]
                lpi = lp_v[pl.ds(c * R + i, LANES)][0]
                for a in range(NSEG):
                    r = i * NSEG + a
                    for j in range(SUB // LANES):
                        sl = pl.ds(j * LANES, LANES)
                        pe = pl.ds(a * SUB + j * LANES, LANES)
                        rows_v[b, r, sl] = (rows_v[b, r, sl]
                                            + fpi * pe_v[0, pe]
                                            + lpi * pe_v[1, pe])
                return 0

            lax.fori_loop(0, R, row_body, 0)

        def round_body(m, carry):
            c0 = m * NBUF
            ghs = []
            for b in range(NBUF):
                k0 = (c0 + b) * KPC
                ghs.append(pltpu.async_copy(
                    tab_sh.at[idx_v.at[pl.ds(k0, KPC)]],
                    rows_v.at[b], gsems[b]))
            ohs = []
            for b in range(NBUF):
                ghs[b].wait()
                compute_chunk(c0 + b, b)
                k0 = (c0 + b) * KPC
                ohs.append(pltpu.async_copy(
                    rows_v.at[b],
                    out_h.at[pl.ds(base_w * NSEG + k0, KPC)], osems[b]))
            for oh in ohs:
                oh.wait()
            return carry

        lax.fori_loop(0, n_rounds, round_body, 0)

    out = sc_kernel(table, coords_bits, fp, lp,
                    first_page_embedding, last_page_embedding)
    return out.reshape(B, L, SIZE)


# speedup vs baseline: 2.4643x; 1.0021x over previous
import functools

import jax
import jax.numpy as jnp
from jax import lax
from jax.experimental import pallas as pl
from jax.experimental.pallas import tpu as pltpu
from jax.experimental.pallas import tpu_sc as plsc

N_POS = 1024
SUB = 128
SIZE = 768
NSEG = 6
LANES = 16


def _sc_workers():
    try:
        info = plsc.get_sparse_core_info()
        return info.num_cores, info.num_subcores
    except Exception:
        return 2, 16


def kernel(xmin, ymin, xmax, ymax, width, height, first_page, last_page,
           x_table, y_table, w_table, h_table,
           first_page_embedding, last_page_embedding):
    B, L = xmin.shape
    NB = B * L
    NC, NS = _sc_workers()
    NW = NC * NS
    rows_per_w = NB // NW
    R = 20
    NBUF = 4
    n_chunks = rows_per_w // R
    n_rounds = n_chunks // NBUF
    KPC = R * NSEG
    KW = rows_per_w * NSEG
    assert rows_per_w * NW == NB and n_rounds * NBUF * R == rows_per_w

    table = jnp.concatenate([x_table, y_table, w_table, h_table], axis=0)

    coords_i = jnp.stack(
        [a.reshape(NB) for a in (xmin, ymin, xmax, ymax, width, height)],
        axis=-1).reshape(NB * NSEG)
    coords_bits = lax.bitcast_convert_type(coords_i, jnp.int32)
    fp = first_page.reshape(NB)
    lp = last_page.reshape(NB)

    mesh = plsc.VectorSubcoreMesh(core_axis_name="c", subcore_axis_name="s",
                                  num_cores=NC, num_subcores=NS)

    @functools.partial(
        pl.kernel,
        out_type=jax.ShapeDtypeStruct((NB * NSEG, SUB), jnp.float32),
        mesh=mesh,
        scratch_types=[
            pltpu.VMEM((KW,), jnp.int32),
            pltpu.VMEM((rows_per_w + LANES,), jnp.float32),
            pltpu.VMEM((rows_per_w + LANES,), jnp.float32),
            pltpu.VMEM((NBUF, KPC, SUB), jnp.float32),
            pltpu.VMEM_SHARED((4 * N_POS, SUB), jnp.float32),
            pltpu.VMEM((2, SIZE), jnp.float32),
            pltpu.SemaphoreType.DMA,
            pltpu.SemaphoreType.DMA,
            pltpu.SemaphoreType.DMA,
            pltpu.SemaphoreType.DMA,
            pltpu.SemaphoreType.DMA,
            pltpu.SemaphoreType.DMA,
            pltpu.SemaphoreType.DMA,
            pltpu.SemaphoreType.DMA,
        ],
    )
    def sc_kernel(table_h, coords_h, fp_h, lp_h, fpe_h, lpe_h, out_h,
                  idx_v, fp_v, lp_v, rows_v, tab_sh, pe_v, *sems):
        gsems, osems = sems[:NBUF], sems[NBUF:]
        lane_iota = lax.broadcasted_iota(jnp.int32, (LANES,), 0)
        lane_bcast = [lane_iota * 0 + u for u in range(LANES)]
        sid = lax.axis_index("s")
        wid = sid * NC + lax.axis_index("c")
        base_w = wid * rows_per_w
        @pl.when(sid == 0)
        def _stage_table():
            pltpu.sync_copy(table_h, tab_sh)
        plsc.subcore_barrier()
        pltpu.sync_copy(fpe_h, pe_v.at[0])
        pltpu.sync_copy(lpe_h, pe_v.at[1])
        pltpu.sync_copy(fp_h.at[pl.ds(base_w, rows_per_w)],
                        fp_v.at[pl.ds(0, rows_per_w)])
        pltpu.sync_copy(lp_h.at[pl.ds(base_w, rows_per_w)],
                        lp_v.at[pl.ds(0, rows_per_w)])
        pltpu.sync_copy(coords_h.at[pl.ds(base_w * NSEG, KW)], idx_v)

        def idx_body(t, _):
            sl = pl.ds(t * LANES, LANES)
            k = t * LANES + lane_iota
            s = k - 6 * ((k * 43691) >> 18)
            v = lax.bitcast_convert_type(idx_v[sl], jnp.float32)
            v = jnp.where(s == 5, v * 5.0, v)
            v = jnp.minimum(v * float(N_POS), float(N_POS - 1))
            off = jnp.where(s >= 4, s - 2, s & 1) * N_POS
            idx_v[sl] = v.astype(jnp.int32) + off
            return 0

        lax.fori_loop(0, KW // LANES, idx_body, 0)

        def compute_chunk(c, b):
            def row_body(i, _):
                fpi = fp_v[pl.ds(c * R + i, LANES)][0]
                lpi = lp_v[pl.ds(c * R + i, LANES)][0]
                for a in range(NSEG):
                    r = i * NSEG + a
                    for j in range(SUB // LANES):
                        sl = pl.ds(j * LANES, LANES)
                        pe = pl.ds(a * SUB + j * LANES, LANES)
                        rows_v[b, r, sl] = (rows_v[b, r, sl]
                                            + fpi * pe_v[0, pe]
                                            + lpi * pe_v[1, pe])
                return 0

            lax.fori_loop(0, R, row_body, 0)

        def round_body(m, carry):
            c0 = m * NBUF
            ghs = []
            for b in range(NBUF):
                k0 = (c0 + b) * KPC
                ghs.append(pltpu.async_copy(
                    tab_sh.at[idx_v.at[pl.ds(k0, KPC)]],
                    rows_v.at[b], gsems[b]))
            ohs = []
            for b in range(NBUF):
                ghs[b].wait()
                compute_chunk(c0 + b, b)
                k0 = (c0 + b) * KPC
                ohs.append(pltpu.async_copy(
                    rows_v.at[b],
                    out_h.at[pl.ds(base_w * NSEG + k0, KPC)], osems[b]))
            for oh in ohs:
                oh.wait()
            return carry

        lax.fori_loop(0, n_rounds, round_body, 0)

    out = sc_kernel(table, coords_bits, fp, lp,
                    first_page_embedding, last_page_embedding)
    return out.reshape(B, L, SIZE)


# speedup vs baseline: 2.5499x; 1.0347x over previous
import functools

import jax
import jax.numpy as jnp
from jax import lax
from jax.experimental import pallas as pl
from jax.experimental.pallas import tpu as pltpu
from jax.experimental.pallas import tpu_sc as plsc

N_POS = 1024
SUB = 128
SIZE = 768
NSEG = 6
LANES = 16


def _sc_workers():
    try:
        info = plsc.get_sparse_core_info()
        return info.num_cores, info.num_subcores
    except Exception:
        return 2, 16


def kernel(xmin, ymin, xmax, ymax, width, height, first_page, last_page,
           x_table, y_table, w_table, h_table,
           first_page_embedding, last_page_embedding):
    B, L = xmin.shape
    NB = B * L
    NC, NS = _sc_workers()
    NW = NC * NS
    rows_per_w = NB // NW
    R = 20
    NBUF = 4
    n_chunks = rows_per_w // R
    n_rounds = n_chunks // NBUF
    KPC = R * NSEG
    KW = rows_per_w * NSEG
    assert rows_per_w * NW == NB and n_rounds * NBUF * R == rows_per_w

    coords = [a.reshape(NB)
              for a in (xmin, ymin, xmax, ymax, width, height)]
    fp = first_page.reshape(NB)
    lp = last_page.reshape(NB)

    mesh = plsc.VectorSubcoreMesh(core_axis_name="c", subcore_axis_name="s",
                                  num_cores=NC, num_subcores=NS)

    @functools.partial(
        pl.kernel,
        out_type=jax.ShapeDtypeStruct((NB * NSEG, SUB), jnp.float32),
        mesh=mesh,
        scratch_types=[
            pltpu.VMEM((KW,), jnp.int32),
            pltpu.VMEM((rows_per_w + LANES,), jnp.float32),
            pltpu.VMEM((rows_per_w + LANES,), jnp.int32),
            pltpu.VMEM((rows_per_w + LANES,), jnp.float32),
            pltpu.VMEM((rows_per_w + LANES,), jnp.float32),
            pltpu.VMEM((NBUF, KPC, SUB), jnp.float32),
            pltpu.VMEM_SHARED((4 * N_POS, SUB), jnp.float32),
            pltpu.VMEM((2, SIZE), jnp.float32),
            pltpu.SemaphoreType.DMA,
            pltpu.SemaphoreType.DMA,
            pltpu.SemaphoreType.DMA,
            pltpu.SemaphoreType.DMA,
            pltpu.SemaphoreType.DMA,
            pltpu.SemaphoreType.DMA,
            pltpu.SemaphoreType.DMA,
            pltpu.SemaphoreType.DMA,
        ],
    )
    def sc_kernel(xt_h, yt_h, wt_h, ht_h,
                  xmin_h, ymin_h, xmax_h, ymax_h, w_h, h_h,
                  fp_h, lp_h, fpe_h, lpe_h, out_h,
                  idx_v, seg_v, segi_v, fp_v, lp_v, rows_v, tab_sh, pe_v,
                  *sems):
        gsems, osems = sems[:NBUF], sems[NBUF:]
        lane_iota = lax.broadcasted_iota(jnp.int32, (LANES,), 0)
        sid = lax.axis_index("s")
        wid = sid * NC + lax.axis_index("c")
        base_w = wid * rows_per_w

        @pl.when(sid == 0)
        def _stage_tables():
            pltpu.sync_copy(xt_h, tab_sh.at[pl.ds(0, N_POS)])
            pltpu.sync_copy(yt_h, tab_sh.at[pl.ds(N_POS, N_POS)])
            pltpu.sync_copy(wt_h, tab_sh.at[pl.ds(2 * N_POS, N_POS)])
            pltpu.sync_copy(ht_h, tab_sh.at[pl.ds(3 * N_POS, N_POS)])
        plsc.subcore_barrier()

        pltpu.sync_copy(fpe_h, pe_v.at[0])
        pltpu.sync_copy(lpe_h, pe_v.at[1])
        pltpu.sync_copy(fp_h.at[pl.ds(base_w, rows_per_w)],
                        fp_v.at[pl.ds(0, rows_per_w)])
        pltpu.sync_copy(lp_h.at[pl.ds(base_w, rows_per_w)],
                        lp_v.at[pl.ds(0, rows_per_w)])

        coord_hs = [xmin_h, ymin_h, xmax_h, ymax_h, w_h, h_h]
        OFF = (0, N_POS, 0, N_POS, 2 * N_POS, 3 * N_POS)

        perms = []
        masks = []
        for j in range(3):
            lj = 16 * j + lane_iota
            dj = (lj * 43691) >> 18
            sj = lj - 6 * dj
            perms.append(dj)
            masks.append(sj)

        for a in range(NSEG):
            pltpu.sync_copy(coord_hs[a].at[pl.ds(base_w, rows_per_w)],
                            seg_v.at[pl.ds(0, rows_per_w)])

            def disc_body(t, _, a=a):
                sl = pl.ds(t * LANES, LANES)
                v = seg_v[sl]
                if a == 5:
                    v = v * 5.0
                v = jnp.minimum(v * float(N_POS), float(N_POS - 1))
                segi_v[sl] = v.astype(jnp.int32) + OFF[a]
                return 0

            lax.fori_loop(0, rows_per_w // LANES, disc_body, 0)

            def ilv_body(g, _, a=a):
                src = segi_v[pl.ds(g * 8, LANES)]
                for j in range(3):
                    osl = pl.ds(g * 48 + j * LANES, LANES)
                    vals = jnp.take_along_axis(src, perms[j], axis=0)
                    if a == 0:
                        idx_v[osl] = vals
                    else:
                        idx_v[osl] = jnp.where(masks[j] == a, vals,
                                               idx_v[osl])
                return 0

            lax.fori_loop(0, rows_per_w // 8, ilv_body, 0)

        def compute_chunk(c, b):
            def row_body(i, _):
                fpi = fp_v[pl.ds(c * R + i, LANES)][0]
                lpi = lp_v[pl.ds(c * R + i, LANES)][0]
                for a in range(NSEG):
                    r = i * NSEG + a
                    for j in range(SUB // LANES):
                        sl = pl.ds(j * LANES, LANES)
                        pe = pl.ds(a * SUB + j * LANES, LANES)
                        rows_v[b, r, sl] = (rows_v[b, r, sl]
                                            + fpi * pe_v[0, pe]
                                            + lpi * pe_v[1, pe])
                return 0

            lax.fori_loop(0, R, row_body, 0)

        def round_body(m, carry):
            c0 = m * NBUF
            ghs = []
            for b in range(NBUF):
                k0 = (c0 + b) * KPC
                ghs.append(pltpu.async_copy(
                    tab_sh.at[idx_v.at[pl.ds(k0, KPC)]],
                    rows_v.at[b], gsems[b]))
            ohs = []
            for b in range(NBUF):
                ghs[b].wait()
                compute_chunk(c0 + b, b)
                k0 = (c0 + b) * KPC
                ohs.append(pltpu.async_copy(
                    rows_v.at[b],
                    out_h.at[pl.ds(base_w * NSEG + k0, KPC)], osems[b]))
            for oh in ohs:
                oh.wait()
            return carry

        lax.fori_loop(0, n_rounds, round_body, 0)

    out = sc_kernel(x_table, y_table, w_table, h_table,
                    *coords, fp, lp,
                    first_page_embedding, last_page_embedding)
    return out.reshape(B, L, SIZE)


# speedup vs baseline: 2.5554x; 1.0021x over previous
import functools

import jax
import jax.numpy as jnp
from jax import lax
from jax.experimental import pallas as pl
from jax.experimental.pallas import tpu as pltpu
from jax.experimental.pallas import tpu_sc as plsc

N_POS = 1024
SUB = 128
SIZE = 768
NSEG = 6
LANES = 16


def _sc_workers():
    try:
        info = plsc.get_sparse_core_info()
        return info.num_cores, info.num_subcores
    except Exception:
        return 2, 16


def kernel(xmin, ymin, xmax, ymax, width, height, first_page, last_page,
           x_table, y_table, w_table, h_table,
           first_page_embedding, last_page_embedding):
    B, L = xmin.shape
    NB = B * L
    NC, NS = _sc_workers()
    NW = NC * NS
    rows_per_w = NB // NW
    R = 20
    NBUF = 4
    n_chunks = rows_per_w // R
    n_rounds = n_chunks // NBUF
    KPC = R * NSEG
    KW = rows_per_w * NSEG
    assert rows_per_w * NW == NB and n_rounds * NBUF * R == rows_per_w

    coords = [a.reshape(NB)
              for a in (xmin, ymin, xmax, ymax, width, height)]
    fp = first_page.reshape(NB)
    lp = last_page.reshape(NB)

    mesh = plsc.VectorSubcoreMesh(core_axis_name="c", subcore_axis_name="s",
                                  num_cores=NC, num_subcores=NS)

    @functools.partial(
        pl.kernel,
        out_type=jax.ShapeDtypeStruct((NB * NSEG, SUB), jnp.float32),
        mesh=mesh,
        scratch_types=[
            pltpu.VMEM((KW,), jnp.int32),
            pltpu.VMEM((rows_per_w + LANES,), jnp.float32),
            pltpu.VMEM((rows_per_w + LANES,), jnp.int32),
            pltpu.VMEM((rows_per_w + LANES,), jnp.float32),
            pltpu.VMEM((rows_per_w + LANES,), jnp.float32),
            pltpu.VMEM((KPC, SUB), jnp.float32),
            pltpu.VMEM((KPC, SUB), jnp.float32),
            pltpu.VMEM((KPC, SUB), jnp.float32),
            pltpu.VMEM((KPC, SUB), jnp.float32),
            pltpu.VMEM_SHARED((4 * N_POS, SUB), jnp.float32),
            pltpu.VMEM((2, SIZE), jnp.float32),
            pltpu.SemaphoreType.DMA,
            pltpu.SemaphoreType.DMA,
            pltpu.SemaphoreType.DMA,
            pltpu.SemaphoreType.DMA,
            pltpu.SemaphoreType.DMA,
            pltpu.SemaphoreType.DMA,
            pltpu.SemaphoreType.DMA,
            pltpu.SemaphoreType.DMA,
        ],
    )
    def sc_kernel(xt_h, yt_h, wt_h, ht_h,
                  xmin_h, ymin_h, xmax_h, ymax_h, w_h, h_h,
                  fp_h, lp_h, fpe_h, lpe_h, out_h,
                  idx_v, seg_v, segi_v, fp_v, lp_v,
                  rows0_v, rows1_v, rows2_v, rows3_v, tab_sh, pe_v,
                  *sems):
        rows_bufs = [rows0_v, rows1_v, rows2_v, rows3_v]
        gsems, osems = sems[:NBUF], sems[NBUF:]
        lane_iota = lax.broadcasted_iota(jnp.int32, (LANES,), 0)
        sid = lax.axis_index("s")
        wid = sid * NC + lax.axis_index("c")
        base_w = wid * rows_per_w

        @pl.when(sid == 0)
        def _stage_tables():
            pltpu.sync_copy(xt_h, tab_sh.at[pl.ds(0, N_POS)])
            pltpu.sync_copy(yt_h, tab_sh.at[pl.ds(N_POS, N_POS)])
            pltpu.sync_copy(wt_h, tab_sh.at[pl.ds(2 * N_POS, N_POS)])
            pltpu.sync_copy(ht_h, tab_sh.at[pl.ds(3 * N_POS, N_POS)])
        plsc.subcore_barrier()

        pltpu.sync_copy(fpe_h, pe_v.at[0])
        pltpu.sync_copy(lpe_h, pe_v.at[1])
        pltpu.sync_copy(fp_h.at[pl.ds(base_w, rows_per_w)],
                        fp_v.at[pl.ds(0, rows_per_w)])
        pltpu.sync_copy(lp_h.at[pl.ds(base_w, rows_per_w)],
                        lp_v.at[pl.ds(0, rows_per_w)])

        coord_hs = [xmin_h, ymin_h, xmax_h, ymax_h, w_h, h_h]
        OFF = (0, N_POS, 0, N_POS, 2 * N_POS, 3 * N_POS)

        perms = []
        masks = []
        for j in range(3):
            lj = 16 * j + lane_iota
            dj = (lj * 43691) >> 18
            sj = lj - 6 * dj
            perms.append(dj)
            masks.append(sj)

        for a in range(NSEG):
            pltpu.sync_copy(coord_hs[a].at[pl.ds(base_w, rows_per_w)],
                            seg_v.at[pl.ds(0, rows_per_w)])

            def disc_body(t, _, a=a):
                sl = pl.ds(t * LANES, LANES)
                v = seg_v[sl]
                if a == 5:
                    v = v * 5.0
                v = jnp.minimum(v * float(N_POS), float(N_POS - 1))
                segi_v[sl] = v.astype(jnp.int32) + OFF[a]
                return 0

            lax.fori_loop(0, rows_per_w // LANES, disc_body, 0)

            def ilv_body(g, _, a=a):
                src = segi_v[pl.ds(g * 8, LANES)]
                for j in range(3):
                    osl = pl.ds(g * 48 + j * LANES, LANES)
                    vals = jnp.take_along_axis(src, perms[j], axis=0)
                    if a == 0:
                        idx_v[osl] = vals
                    else:
                        idx_v[osl] = jnp.where(masks[j] == a, vals,
                                               idx_v[osl])
                return 0

            lax.fori_loop(0, rows_per_w // 8, ilv_body, 0)

        def compute_chunk(c, rv):
            def row_body(i, _):
                fpi = fp_v[pl.ds(c * R + i, LANES)][0]
                lpi = lp_v[pl.ds(c * R + i, LANES)][0]
                for a in range(NSEG):
                    r = i * NSEG + a
                    for j in range(SUB // LANES):
                        sl = pl.ds(j * LANES, LANES)
                        pe = pl.ds(a * SUB + j * LANES, LANES)
                        rv[r, sl] = (rv[r, sl]
                                     + fpi * pe_v[0, pe]
                                     + lpi * pe_v[1, pe])
                return 0

            lax.fori_loop(0, R, row_body, 0)

        base_b = wid * (rows_per_w // L)

        def round_body(m, carry):
            c0 = m * NBUF
            ghs = []
            for b in range(NBUF):
                k0 = (c0 + b) * KPC
                ghs.append(pltpu.async_copy(
                    tab_sh.at[idx_v.at[pl.ds(k0, KPC)]],
                    rows_bufs[b], gsems[b]))
            ohs = []
            for b in range(NBUF):
                ghs[b].wait()
                compute_chunk(c0 + b, rows_bufs[b])
                k0 = (c0 + b) * KPC
                ohs.append(pltpu.async_copy(
                    rows_bufs[b],
                    out_h.at[pl.ds(base_w * NSEG + k0, KPC)], osems[b]))
            for oh in ohs:
                oh.wait()
            return carry

        lax.fori_loop(0, n_rounds, round_body, 0)

    out = sc_kernel(x_table, y_table, w_table, h_table,
                    *coords, fp, lp,
                    first_page_embedding, last_page_embedding)
    return out.reshape(B, L, SIZE)
